# baseline (speedup 1.0000x reference)
import functools

import jax
import jax.numpy as jnp
from jax import lax
from jax.experimental import pallas as pl
from jax.experimental.pallas import tpu as pltpu
from jax.experimental.pallas import tpu_sc as plsc

_B = 16384 * 100
_D = 32
_NC, _NS = 2, 16
_NW = _NC * _NS
_BW = _B // _NW
_C = 1024
_NCHUNK = _BW // _C
_GL = 128
_G = _C // _GL
_S0 = 100


@functools.partial(
    pl.kernel,
    out_type=jax.ShapeDtypeStruct((_B, _D), jnp.float32),
    mesh=plsc.VectorSubcoreMesh(
        core_axis_name="c", subcore_axis_name="s",
        num_cores=_NC, num_subcores=_NS),
    compiler_params=pltpu.CompilerParams(use_tc_tiling_on_sc=False),
    scratch_types=[
        pltpu.VMEM((_C,), jnp.int32),
        pltpu.VMEM((_C,), jnp.int32),
        pltpu.VMEM((_C,), jnp.int32),
        pltpu.VMEM((_C, _D), jnp.float32),
        pltpu.SemaphoreType.DMA,
    ],
)
def _embed_gather(mi_hbm, table_hbm, out_hbm, m0_v, m1_v, idx_v, rows_v, sem):
    wid = lax.axis_index("s") * _NC + lax.axis_index("c")
    wbase = wid * _BW

    @pl.loop(0, _NCHUNK)
    def _chunk(t):
        base = wbase + t * _C
        pltpu.sync_copy(mi_hbm.at[0, pl.ds(base, _C)], m0_v)
        pltpu.sync_copy(mi_hbm.at[1, pl.ds(base, _C)], m1_v)

        @pl.loop(0, _C // 16)
        def _piece(k):
            off = k * 16
            idx_v[pl.ds(off, 16)] = (
                m0_v[pl.ds(off, 16)] * _S0 + m1_v[pl.ds(off, 16)])

        pltpu.async_copy(table_hbm.at[idx_v], rows_v, sem).wait()
        pltpu.sync_copy(rows_v, out_hbm.at[pl.ds(base, _C)])


def kernel(multi_index, table):
    mi = multi_index.reshape(2, -1)
    out = _embed_gather(mi, table)
    return out.reshape(multi_index.shape[1], multi_index.shape[2], _D)


# speedup vs baseline: 4.8197x; 4.8197x over previous
import functools

import jax
import jax.numpy as jnp
from jax import lax
from jax.experimental import pallas as pl
from jax.experimental.pallas import tpu as pltpu
from jax.experimental.pallas import tpu_sc as plsc

_N = 16384
_M = 100
_D = 32
_NC, _NS = 2, 16
_NW = _NC * _NS
_RW = _N // _NW
_CR = 16
_NCHUNK = _RW // _CR
_S0 = 100


@functools.partial(
    pl.kernel,
    out_type=jax.ShapeDtypeStruct((_N, _M, _D), jnp.float32),
    mesh=plsc.VectorSubcoreMesh(
        core_axis_name="c", subcore_axis_name="s",
        num_cores=_NC, num_subcores=_NS),
    compiler_params=pltpu.CompilerParams(use_tc_tiling_on_sc=False),
    scratch_types=[
        pltpu.VMEM((_CR, _M), jnp.int32),
        pltpu.VMEM((_CR, _M), jnp.int32),
        pltpu.VMEM((_CR, _M), jnp.int32),
        pltpu.VMEM((_CR, _M, _D), jnp.float32),
        pltpu.SemaphoreType.DMA,
    ],
)
def _embed_gather(mi_hbm, table_hbm, out_hbm, m0_v, m1_v, idx_v, rows_v, sem):
    wid = lax.axis_index("s") * _NC + lax.axis_index("c")
    wbase = wid * _RW

    @pl.loop(0, _NCHUNK)
    def _chunk(t):
        r0 = wbase + t * _CR
        pltpu.sync_copy(mi_hbm.at[0, pl.ds(r0, _CR)], m0_v)
        pltpu.sync_copy(mi_hbm.at[1, pl.ds(r0, _CR)], m1_v)

        @pl.loop(0, _CR)
        def _row(j):
            for k in (0, 16, 32, 48, 64, 80, _M - 16):
                idx_v[j, pl.ds(k, 16)] = (
                    m0_v[j, pl.ds(k, 16)] * _S0 + m1_v[j, pl.ds(k, 16)])

        copies = [
            pltpu.async_copy(table_hbm.at[idx_v.at[j]], rows_v.at[j], sem)
            for j in range(_CR)
        ]
        for cp in copies:
            cp.wait()
        pltpu.sync_copy(rows_v, out_hbm.at[pl.ds(r0, _CR)])


def kernel(multi_index, table):
    return _embed_gather(multi_index, table)
